# Initial kernel scaffold; baseline (speedup 1.0000x reference)
import jax
import jax.numpy as jnp
from jax.experimental import pallas as pl


def kernel(x, edge_index, S, R, W1, b1, W2, b2, Wg1, bg1, Wg2, bg2, W3, b3, W4, b4):
    raise NotImplementedError("write your pallas kernel here")


# speedup vs baseline: 6.0457x; 6.0457x over previous
import functools

import jax
import jax.numpy as jnp
from jax import lax
from jax.experimental import pallas as pl
from jax.experimental.pallas import tpu as pltpu
from jax.experimental.pallas import tpu_sc as plsc

N = 10000
E = 160000
D_FEAT = 256
SUB_REP = 256
HID = 1024
GCN_H = 512
OUT = 128

NB = 1000
GRID = N // NB
NCHUNK = 4
FC = GCN_H // NCHUNK
TILES = 16
E_PER_TILE = E // TILES
EDGE_B = 80
N_BATCH = E_PER_TILE // EDGE_B
ROWS_MAIN = 640
ROWS_LAST = N - (TILES - 1) * ROWS_MAIN


def _per_tile_rows(sid, fn):
    @pl.when(sid < TILES - 1)
    def _():
        fn(pl.ds(pl.multiple_of(sid * ROWS_MAIN, 8), ROWS_MAIN), ROWS_MAIN)

    @pl.when(sid == TILES - 1)
    def _():
        fn(pl.ds((TILES - 1) * ROWS_MAIN, ROWS_LAST), ROWS_LAST)


def _dinv(deg_ref):
    return lax.rsqrt(deg_ref[0, :, :1] + deg_ref[1, :, :1] + 1.0)


def _front_body(deg_ref, x_ref, s_ref, r_ref, w1a_ref, w1b_ref, b1_ref,
                w2_ref, b2_ref, wg1_ref, o0, o1, o2, o3):
    f32 = jnp.float32
    sr = jnp.dot(s_ref[...], r_ref[...], preferred_element_type=f32)
    t = (jnp.dot(x_ref[...], w1a_ref[...], preferred_element_type=f32)
         + jnp.dot(sr, w1b_ref[...], preferred_element_type=f32) + b1_ref[...])
    h1 = jnp.tanh(t)
    h2 = jnp.tanh(jnp.dot(h1, w2_ref[...], preferred_element_type=f32) + b2_ref[...])
    hw = jnp.dot(h2, wg1_ref[...], preferred_element_type=f32)
    dinv = _dinv(deg_ref)
    hws = hw * dinv
    o0[...] = hws[:, 0 * FC:1 * FC]
    o1[...] = hws[:, 1 * FC:2 * FC]
    o2[...] = hws[:, 2 * FC:3 * FC]
    o3[...] = hws[:, 3 * FC:4 * FC]


def _mid_body(deg_ref, p0, p1, p2, p3, h0, h1, h2, h3, wg_ref, bg_ref,
              o0, o1, o2, o3):
    f32 = jnp.float32
    p = jnp.concatenate([p0[...], p1[...], p2[...], p3[...]], axis=1)
    hprev = jnp.concatenate([h0[...], h1[...], h2[...], h3[...]], axis=1)
    dinv = _dinv(deg_ref)
    h = jnp.maximum(dinv * (p + hprev) + bg_ref[...], 0.0)
    hw = jnp.dot(h, wg_ref[...], preferred_element_type=f32) * dinv
    o0[...] = hw[:, 0 * FC:1 * FC]
    o1[...] = hw[:, 1 * FC:2 * FC]
    o2[...] = hw[:, 2 * FC:3 * FC]
    o3[...] = hw[:, 3 * FC:4 * FC]


def _back_body(deg_ref, p0, p1, p2, p3, h0, h1, h2, h3, bg_ref,
               w3_ref, b3_ref, w4_ref, b4_ref, out_ref):
    f32 = jnp.float32
    p = jnp.concatenate([p0[...], p1[...], p2[...], p3[...]], axis=1)
    hprev = jnp.concatenate([h0[...], h1[...], h2[...], h3[...]], axis=1)
    dinv = _dinv(deg_ref)
    h = jnp.maximum(dinv * (p + hprev) + bg_ref[...], 0.0)
    h5 = jnp.tanh(jnp.dot(h, w3_ref[...], preferred_element_type=f32) + b3_ref[...])
    out_ref[...] = jnp.tanh(jnp.dot(h5, w4_ref[...], preferred_element_type=f32)
                            + b4_ref[...])


def _row_spec(cols):
    return pl.BlockSpec((NB, cols), lambda i: (i, 0))


_DEG_SPEC = pl.BlockSpec((2, NB, FC), lambda i: (0, i, 0))


def _full_spec(shape):
    nd = len(shape)
    return pl.BlockSpec(shape, lambda i: (0,) * nd)


def _chunk_out_shapes():
    return [jax.ShapeDtypeStruct((N, FC), jnp.float32) for _ in range(NCHUNK)]


def _front(deg16, x, S, R, W1a, W1b, b1, W2, b2, Wg1):
    return pl.pallas_call(
        _front_body,
        grid=(GRID,),
        in_specs=[
            _DEG_SPEC, _row_spec(D_FEAT), _row_spec(64),
            _full_spec((64, SUB_REP)),
            _full_spec((D_FEAT, HID)), _full_spec((SUB_REP, HID)),
            _full_spec((1, HID)),
            _full_spec((HID, GCN_H)), _full_spec((1, GCN_H)),
            _full_spec((GCN_H, GCN_H)),
        ],
        out_specs=[_row_spec(FC)] * NCHUNK,
        out_shape=_chunk_out_shapes(),
    )(deg16, x, S, R, W1a, W1b, b1, W2, b2, Wg1)


def _mid(deg16, p, h, Wg2, bg1):
    return pl.pallas_call(
        _mid_body,
        grid=(GRID,),
        in_specs=[_DEG_SPEC] + [_row_spec(FC)] * 8 + [
            _full_spec((GCN_H, GCN_H)), _full_spec((1, GCN_H))],
        out_specs=[_row_spec(FC)] * NCHUNK,
        out_shape=_chunk_out_shapes(),
    )(deg16, *p, *h, Wg2, bg1)


def _back(deg16, p, h, bg2, W3, b3, W4, b4):
    return pl.pallas_call(
        _back_body,
        grid=(GRID,),
        in_specs=[_DEG_SPEC] + [_row_spec(FC)] * 8 + [
            _full_spec((1, GCN_H)),
            _full_spec((GCN_H, HID)), _full_spec((1, HID)),
            _full_spec((HID, OUT)), _full_spec((1, OUT))],
        out_specs=_row_spec(OUT),
        out_shape=jax.ShapeDtypeStruct((N, OUT), jnp.float32),
    )(deg16, *p, *h, bg2, W3, b3, W4, b4)


@functools.cache
def _mesh():
    return plsc.VectorSubcoreMesh(core_axis_name="c", subcore_axis_name="s")


DEG_B = 40
E_PER_TILE2 = E // (2 * TILES)
DEG_BATCHES = E_PER_TILE2 // DEG_B


def _deg_body(dst_hbm, ones_hbm, zeros_hbm, out_hbm, idx_v, ones_v, acc):
    cid = lax.axis_index("c")
    sid = lax.axis_index("s")
    pltpu.sync_copy(ones_hbm, ones_v)
    _per_tile_rows(sid, lambda rows, ln:
                   pltpu.sync_copy(zeros_hbm.at[pl.ds(0, ln)], acc.at[rows]))
    plsc.subcore_barrier()

    def step(k, carry):
        base = pl.multiple_of((cid * TILES + sid) * E_PER_TILE2 + k * DEG_B, 8)
        pltpu.sync_copy(dst_hbm.at[pl.ds(base, DEG_B)], idx_v)
        pltpu.sync_copy(ones_v, acc.at[idx_v], add=True)
        return carry

    lax.fori_loop(0, DEG_BATCHES, step, 0)
    plsc.subcore_barrier()

    for core in range(2):
        @pl.when(cid == core)
        def _(core=core):
            _per_tile_rows(sid, lambda rows, ln:
                           pltpu.sync_copy(acc.at[rows], out_hbm.at[core].at[rows]))


def _scatter_body(h0_hbm, h1_hbm, h2_hbm, h3_hbm, src_hbm, dst_hbm, zeros_hbm,
                  o0_hbm, o1_hbm, o2_hbm, o3_hbm,
                  src_v, dst_v, rows_v, acc, sem):
    cid = lax.axis_index("c")
    sid = lax.axis_index("s")
    hw_refs = (h0_hbm, h1_hbm, h2_hbm, h3_hbm)
    out_refs = (o0_hbm, o1_hbm, o2_hbm, o3_hbm)

    for chunk in range(NCHUNK):
        @pl.when(cid == chunk // 2)
        def _(chunk=chunk):
            _per_tile_rows(sid, lambda rows, ln:
                           pltpu.sync_copy(zeros_hbm.at[pl.ds(0, ln)], acc.at[rows]))
            plsc.subcore_barrier()

            def step(k, carry):
                base = pl.multiple_of(sid * E_PER_TILE + k * EDGE_B, 8)
                pltpu.sync_copy(src_hbm.at[pl.ds(base, EDGE_B)], src_v)
                pltpu.sync_copy(dst_hbm.at[pl.ds(base, EDGE_B)], dst_v)
                pltpu.async_copy(hw_refs[chunk].at[src_v], rows_v, sem).wait()
                pltpu.sync_copy(rows_v, acc.at[dst_v], add=True)
                return carry

            lax.fori_loop(0, N_BATCH, step, 0)
            plsc.subcore_barrier()
            _per_tile_rows(sid, lambda rows, ln:
                           pltpu.sync_copy(acc.at[rows], out_refs[chunk].at[rows]))


@functools.cache
def _deg_kernel_fn():
    return pl.kernel(
        _deg_body,
        mesh=_mesh(),
        out_type=jax.ShapeDtypeStruct((2, N, FC), jnp.float32),
        scratch_types=[
            pltpu.VMEM((DEG_B,), jnp.int32),
            pltpu.VMEM((DEG_B, FC), jnp.float32),
            pltpu.VMEM_SHARED((N, FC), jnp.float32),
        ],
    )


@functools.cache
def _scatter_kernel_fn():
    return pl.kernel(
        _scatter_body,
        mesh=_mesh(),
        out_type=[jax.ShapeDtypeStruct((N, FC), jnp.float32) for _ in range(NCHUNK)],
        scratch_types=[
            pltpu.VMEM((EDGE_B,), jnp.int32),
            pltpu.VMEM((EDGE_B,), jnp.int32),
            pltpu.VMEM((EDGE_B, FC), jnp.float32),
            pltpu.VMEM_SHARED((N, FC), jnp.float32),
            pltpu.SemaphoreType.DMA,
        ],
    )


def _deg_kernel(dst, onesfc, zerosfc):
    return _deg_kernel_fn()(dst, onesfc, zerosfc)


def _scatter_kernel(h0, h1, h2, h3, src, dst, zerosfc):
    return _scatter_kernel_fn()(h0, h1, h2, h3, src, dst, zerosfc)


def kernel(x, edge_index, S, R, W1, b1, W2, b2, Wg1, bg1, Wg2, bg2, W3, b3, W4, b4):
    f32 = jnp.float32
    src = edge_index[0].astype(jnp.int32)
    dst = edge_index[1].astype(jnp.int32)
    onesfc = jnp.ones((DEG_B, FC), f32)
    zerosfc = jnp.zeros((ROWS_MAIN, FC), f32)
    b1r = b1.reshape(1, -1)
    b2r = b2.reshape(1, -1)
    bg1r = bg1.reshape(1, -1)
    bg2r = bg2.reshape(1, -1)
    b3r = b3.reshape(1, -1)
    b4r = b4.reshape(1, -1)

    deg16 = _deg_kernel(dst, onesfc, zerosfc)
    h1c = _front(deg16, x, S, R, W1[:D_FEAT], W1[D_FEAT:], b1r, W2, b2r, Wg1)
    p1c = _scatter_kernel(*h1c, src, dst, zerosfc)
    h2c = _mid(deg16, p1c, h1c, Wg2, bg1r)
    p2c = _scatter_kernel(*h2c, src, dst, zerosfc)
    out = _back(deg16, p2c, h2c, bg2r, W3, b3r, W4, b4r)
    return out


# speedup vs baseline: 15.0347x; 2.4868x over previous
import functools

import jax
import jax.numpy as jnp
from jax import lax
from jax.experimental import pallas as pl
from jax.experimental.pallas import tpu as pltpu
from jax.experimental.pallas import tpu_sc as plsc

N = 10000
E = 160000
D_FEAT = 256
SUB_REP = 256
HID = 1024
GCN_H = 512
OUT = 128

NB = 1000
GRID = N // NB
NCHUNK = 4
FC = GCN_H // NCHUNK
TILES = 16
E_PER_TILE = E // TILES
EDGE_B = 40
N_BATCH = E_PER_TILE // EDGE_B
ROWS_MAIN = 640
ROWS_LAST = N - (TILES - 1) * ROWS_MAIN


def _per_tile_rows(sid, fn):
    @pl.when(sid < TILES - 1)
    def _():
        fn(pl.ds(pl.multiple_of(sid * ROWS_MAIN, 8), ROWS_MAIN), ROWS_MAIN)

    @pl.when(sid == TILES - 1)
    def _():
        fn(pl.ds((TILES - 1) * ROWS_MAIN, ROWS_LAST), ROWS_LAST)


def _dinv(deg_ref):
    return lax.rsqrt(deg_ref[0, :, :1] + deg_ref[1, :, :1] + 1.0)


def _front_body(deg_ref, x_ref, s_ref, r_ref, w1a_ref, w1b_ref, b1_ref,
                w2_ref, b2_ref, wg1_ref, o0, o1, o2, o3):
    f32 = jnp.float32
    sr = jnp.dot(s_ref[...], r_ref[...], preferred_element_type=f32)
    t = (jnp.dot(x_ref[...], w1a_ref[...], preferred_element_type=f32)
         + jnp.dot(sr, w1b_ref[...], preferred_element_type=f32) + b1_ref[...])
    h1 = jnp.tanh(t)
    h2 = jnp.tanh(jnp.dot(h1, w2_ref[...], preferred_element_type=f32) + b2_ref[...])
    hw = jnp.dot(h2, wg1_ref[...], preferred_element_type=f32)
    dinv = _dinv(deg_ref)
    hws = hw * dinv
    o0[...] = hws[:, 0 * FC:1 * FC]
    o1[...] = hws[:, 1 * FC:2 * FC]
    o2[...] = hws[:, 2 * FC:3 * FC]
    o3[...] = hws[:, 3 * FC:4 * FC]


def _mid_body(deg_ref, p0, p1, p2, p3, h0, h1, h2, h3, wg_ref, bg_ref,
              o0, o1, o2, o3):
    f32 = jnp.float32
    p = jnp.concatenate([p0[...], p1[...], p2[...], p3[...]], axis=1)
    hprev = jnp.concatenate([h0[...], h1[...], h2[...], h3[...]], axis=1)
    dinv = _dinv(deg_ref)
    h = jnp.maximum(dinv * (p + hprev) + bg_ref[...], 0.0)
    hw = jnp.dot(h, wg_ref[...], preferred_element_type=f32) * dinv
    o0[...] = hw[:, 0 * FC:1 * FC]
    o1[...] = hw[:, 1 * FC:2 * FC]
    o2[...] = hw[:, 2 * FC:3 * FC]
    o3[...] = hw[:, 3 * FC:4 * FC]


def _back_body(deg_ref, p0, p1, p2, p3, h0, h1, h2, h3, bg_ref,
               w3_ref, b3_ref, w4_ref, b4_ref, out_ref):
    f32 = jnp.float32
    p = jnp.concatenate([p0[...], p1[...], p2[...], p3[...]], axis=1)
    hprev = jnp.concatenate([h0[...], h1[...], h2[...], h3[...]], axis=1)
    dinv = _dinv(deg_ref)
    h = jnp.maximum(dinv * (p + hprev) + bg_ref[...], 0.0)
    h5 = jnp.tanh(jnp.dot(h, w3_ref[...], preferred_element_type=f32) + b3_ref[...])
    out_ref[...] = jnp.tanh(jnp.dot(h5, w4_ref[...], preferred_element_type=f32)
                            + b4_ref[...])


def _row_spec(cols):
    return pl.BlockSpec((NB, cols), lambda i: (i, 0))


_DEG_SPEC = pl.BlockSpec((2, NB, FC), lambda i: (0, i, 0))


def _full_spec(shape):
    nd = len(shape)
    return pl.BlockSpec(shape, lambda i: (0,) * nd)


def _chunk_out_shapes():
    return [jax.ShapeDtypeStruct((N, FC), jnp.float32) for _ in range(NCHUNK)]


def _front(deg16, x, S, R, W1a, W1b, b1, W2, b2, Wg1):
    return pl.pallas_call(
        _front_body,
        grid=(GRID,),
        in_specs=[
            _DEG_SPEC, _row_spec(D_FEAT), _row_spec(64),
            _full_spec((64, SUB_REP)),
            _full_spec((D_FEAT, HID)), _full_spec((SUB_REP, HID)),
            _full_spec((1, HID)),
            _full_spec((HID, GCN_H)), _full_spec((1, GCN_H)),
            _full_spec((GCN_H, GCN_H)),
        ],
        out_specs=[_row_spec(FC)] * NCHUNK,
        out_shape=_chunk_out_shapes(),
    )(deg16, x, S, R, W1a, W1b, b1, W2, b2, Wg1)


def _mid(deg16, p, h, Wg2, bg1):
    return pl.pallas_call(
        _mid_body,
        grid=(GRID,),
        in_specs=[_DEG_SPEC] + [_row_spec(FC)] * 8 + [
            _full_spec((GCN_H, GCN_H)), _full_spec((1, GCN_H))],
        out_specs=[_row_spec(FC)] * NCHUNK,
        out_shape=_chunk_out_shapes(),
    )(deg16, *p, *h, Wg2, bg1)


def _back(deg16, p, h, bg2, W3, b3, W4, b4):
    return pl.pallas_call(
        _back_body,
        grid=(GRID,),
        in_specs=[_DEG_SPEC] + [_row_spec(FC)] * 8 + [
            _full_spec((1, GCN_H)),
            _full_spec((GCN_H, HID)), _full_spec((1, HID)),
            _full_spec((HID, OUT)), _full_spec((1, OUT))],
        out_specs=_row_spec(OUT),
        out_shape=jax.ShapeDtypeStruct((N, OUT), jnp.float32),
    )(deg16, *p, *h, bg2, W3, b3, W4, b4)


@functools.cache
def _mesh():
    return plsc.VectorSubcoreMesh(core_axis_name="c", subcore_axis_name="s")


DEG_B = 40
E_PER_TILE2 = E // (2 * TILES)
DEG_BATCHES = E_PER_TILE2 // DEG_B


def _deg_body(dst_hbm, ones_hbm, zeros_hbm, out_hbm, idx_v, ones_v, acc):
    cid = lax.axis_index("c")
    sid = lax.axis_index("s")
    pltpu.sync_copy(ones_hbm, ones_v)
    _per_tile_rows(sid, lambda rows, ln:
                   pltpu.sync_copy(zeros_hbm.at[pl.ds(0, ln)], acc.at[rows]))
    plsc.subcore_barrier()

    def step(k, carry):
        base = pl.multiple_of((cid * TILES + sid) * E_PER_TILE2 + k * DEG_B, 8)
        pltpu.sync_copy(dst_hbm.at[pl.ds(base, DEG_B)], idx_v)
        pltpu.sync_copy(ones_v, acc.at[idx_v], add=True)
        return carry

    lax.fori_loop(0, DEG_BATCHES, step, 0)
    plsc.subcore_barrier()

    for core in range(2):
        @pl.when(cid == core)
        def _(core=core):
            _per_tile_rows(sid, lambda rows, ln:
                           pltpu.sync_copy(acc.at[rows], out_hbm.at[core].at[rows]))


NBUF = 5


def _scatter_body(h0_hbm, h1_hbm, h2_hbm, h3_hbm, src_hbm, dst_hbm, zeros_hbm,
                  o0_hbm, o1_hbm, o2_hbm, o3_hbm,
                  src_all, dsti, rows, acc, gsem, dsem, ssem):
    cid = lax.axis_index("c")
    sid = lax.axis_index("s")
    hw_refs = (h0_hbm, h1_hbm, h2_hbm, h3_hbm)
    out_refs = (o0_hbm, o1_hbm, o2_hbm, o3_hbm)

    ebase = pl.multiple_of(sid * E_PER_TILE, 8)
    pltpu.sync_copy(src_hbm.at[pl.ds(ebase, E_PER_TILE)], src_all)

    def _start_d(k, b):
        base = pl.multiple_of(sid * E_PER_TILE + k * EDGE_B, 8)
        pltpu.async_copy(dst_hbm.at[pl.ds(base, EDGE_B)], dsti[b], dsem[b])

    def _wait_d(b):
        pltpu.make_async_copy(dst_hbm.at[pl.ds(0, EDGE_B)], dsti[b], dsem[b]).wait()

    def _start_g(k, b, hw_ref):
        off = pl.multiple_of(k * EDGE_B, 8)
        pltpu.async_copy(hw_ref.at[src_all.at[pl.ds(off, EDGE_B)]], rows[b], gsem[b])

    def _wait_g(b, hw_ref):
        pltpu.make_async_copy(hw_ref.at[src_all.at[pl.ds(0, EDGE_B)]],
                              rows[b], gsem[b]).wait()

    def _start_s(b):
        pltpu.async_copy(rows[b], acc.at[dsti[b]], ssem[b], add=True)

    def _wait_s(b):
        pltpu.make_async_copy(rows[b], acc.at[dsti[b]], ssem[b]).wait()

    for chunk in range(NCHUNK):
        @pl.when(cid == chunk // 2)
        def _(chunk=chunk):
            hw_ref = hw_refs[chunk]
            _per_tile_rows(sid, lambda rowsl, ln:
                           pltpu.sync_copy(zeros_hbm.at[pl.ds(0, ln)], acc.at[rowsl]))
            plsc.subcore_barrier()

            for b in range(NBUF):
                _start_d(b, b)
                _start_g(b, b, hw_ref)

            def outer(g, carry):
                for b in range(NBUF):
                    k = g * NBUF + b
                    bw = (b - 1) % NBUF

                    @pl.when((k >= 1) & (k + NBUF - 1 < N_BATCH))
                    def _():
                        _wait_s(bw)
                        _start_d(k + NBUF - 1, bw)
                        _start_g(k + NBUF - 1, bw, hw_ref)

                    _wait_g(b, hw_ref)
                    _wait_d(b)
                    _start_s(b)
                return carry

            lax.fori_loop(0, N_BATCH // NBUF, outer, 0)
            for b in range(NBUF):
                _wait_s(b)
            plsc.subcore_barrier()
            _per_tile_rows(sid, lambda rowsl, ln:
                           pltpu.sync_copy(acc.at[rowsl], out_refs[chunk].at[rowsl]))


@functools.cache
def _deg_kernel_fn():
    return pl.kernel(
        _deg_body,
        mesh=_mesh(),
        out_type=jax.ShapeDtypeStruct((2, N, FC), jnp.float32),
        scratch_types=[
            pltpu.VMEM((DEG_B,), jnp.int32),
            pltpu.VMEM((DEG_B, FC), jnp.float32),
            pltpu.VMEM_SHARED((N, FC), jnp.float32),
        ],
    )


@functools.cache
def _scatter_kernel_fn():
    return pl.kernel(
        _scatter_body,
        mesh=_mesh(),
        out_type=[jax.ShapeDtypeStruct((N, FC), jnp.float32) for _ in range(NCHUNK)],
        scratch_types=[
            pltpu.VMEM((E_PER_TILE,), jnp.int32),
            [pltpu.VMEM((EDGE_B,), jnp.int32) for _ in range(NBUF)],
            [pltpu.VMEM((EDGE_B, FC), jnp.float32) for _ in range(NBUF)],
            pltpu.VMEM_SHARED((N, FC), jnp.float32),
            [pltpu.SemaphoreType.DMA for _ in range(NBUF)],
            [pltpu.SemaphoreType.DMA for _ in range(NBUF)],
            [pltpu.SemaphoreType.DMA for _ in range(NBUF)],
        ],
    )


def _deg_kernel(dst, onesfc, zerosfc):
    return _deg_kernel_fn()(dst, onesfc, zerosfc)


def _scatter_kernel(h0, h1, h2, h3, src, dst, zerosfc):
    return _scatter_kernel_fn()(h0, h1, h2, h3, src, dst, zerosfc)


def kernel(x, edge_index, S, R, W1, b1, W2, b2, Wg1, bg1, Wg2, bg2, W3, b3, W4, b4):
    f32 = jnp.float32
    src = edge_index[0].astype(jnp.int32)
    dst = edge_index[1].astype(jnp.int32)
    onesfc = jnp.ones((DEG_B, FC), f32)
    zerosfc = jnp.zeros((ROWS_MAIN, FC), f32)
    b1r = b1.reshape(1, -1)
    b2r = b2.reshape(1, -1)
    bg1r = bg1.reshape(1, -1)
    bg2r = bg2.reshape(1, -1)
    b3r = b3.reshape(1, -1)
    b4r = b4.reshape(1, -1)

    deg16 = _deg_kernel(dst, onesfc, zerosfc)
    h1c = _front(deg16, x, S, R, W1[:D_FEAT], W1[D_FEAT:], b1r, W2, b2r, Wg1)
    p1c = _scatter_kernel(*h1c, src, dst, zerosfc)
    h2c = _mid(deg16, p1c, h1c, Wg2, bg1r)
    p2c = _scatter_kernel(*h2c, src, dst, zerosfc)
    out = _back(deg16, p2c, h2c, bg2r, W3, b3r, W4, b4r)
    return out


# speedup vs baseline: 16.4921x; 1.0969x over previous
import functools

import jax
import jax.numpy as jnp
from jax import lax
from jax.experimental import pallas as pl
from jax.experimental.pallas import tpu as pltpu
from jax.experimental.pallas import tpu_sc as plsc

N = 10000
E = 160000
D_FEAT = 256
SUB_REP = 256
HID = 1024
GCN_H = 512
OUT = 128

NB = 1000
GRID = N // NB
NCHUNK = 4
FC = GCN_H // NCHUNK
TILES = 16
E_PER_TILE = E // TILES
EDGE_B = 40
N_BATCH = E_PER_TILE // EDGE_B
ROWS_MAIN = 640
ROWS_LAST = N - (TILES - 1) * ROWS_MAIN


def _per_tile_rows(sid, fn):
    @pl.when(sid < TILES - 1)
    def _():
        fn(pl.ds(pl.multiple_of(sid * ROWS_MAIN, 8), ROWS_MAIN), ROWS_MAIN)

    @pl.when(sid == TILES - 1)
    def _():
        fn(pl.ds((TILES - 1) * ROWS_MAIN, ROWS_LAST), ROWS_LAST)


def _dinv(deg_ref):
    return lax.rsqrt(deg_ref[0, :, :1] + deg_ref[1, :, :1] + 1.0)


def _front_body(deg_ref, x_ref, s_ref, r_ref, w1a_ref, w1b_ref, b1_ref,
                w2_ref, b2_ref, wg1_ref, o0, o1, o2, o3):
    f32 = jnp.float32
    sr = jnp.dot(s_ref[...], r_ref[...], preferred_element_type=f32)
    t = (jnp.dot(x_ref[...], w1a_ref[...], preferred_element_type=f32)
         + jnp.dot(sr, w1b_ref[...], preferred_element_type=f32) + b1_ref[...])
    h1 = jnp.tanh(t)
    h2 = jnp.tanh(jnp.dot(h1, w2_ref[...], preferred_element_type=f32) + b2_ref[...])
    hw = jnp.dot(h2, wg1_ref[...], preferred_element_type=f32)
    dinv = _dinv(deg_ref)
    hws = hw * dinv
    o0[...] = hws[:, 0 * FC:1 * FC]
    o1[...] = hws[:, 1 * FC:2 * FC]
    o2[...] = hws[:, 2 * FC:3 * FC]
    o3[...] = hws[:, 3 * FC:4 * FC]


def _mid_body(deg_ref, p0, p1, p2, p3, h0, h1, h2, h3, wg_ref, bg_ref,
              o0, o1, o2, o3):
    f32 = jnp.float32
    p = jnp.concatenate([p0[...], p1[...], p2[...], p3[...]], axis=1)
    hprev = jnp.concatenate([h0[...], h1[...], h2[...], h3[...]], axis=1)
    dinv = _dinv(deg_ref)
    h = jnp.maximum(dinv * (p + hprev) + bg_ref[...], 0.0)
    hw = jnp.dot(h, wg_ref[...], preferred_element_type=f32) * dinv
    o0[...] = hw[:, 0 * FC:1 * FC]
    o1[...] = hw[:, 1 * FC:2 * FC]
    o2[...] = hw[:, 2 * FC:3 * FC]
    o3[...] = hw[:, 3 * FC:4 * FC]


def _back_body(deg_ref, p0, p1, p2, p3, h0, h1, h2, h3, bg_ref,
               w3_ref, b3_ref, w4_ref, b4_ref, out_ref):
    f32 = jnp.float32
    p = jnp.concatenate([p0[...], p1[...], p2[...], p3[...]], axis=1)
    hprev = jnp.concatenate([h0[...], h1[...], h2[...], h3[...]], axis=1)
    dinv = _dinv(deg_ref)
    h = jnp.maximum(dinv * (p + hprev) + bg_ref[...], 0.0)
    h5 = jnp.tanh(jnp.dot(h, w3_ref[...], preferred_element_type=f32) + b3_ref[...])
    out_ref[...] = jnp.tanh(jnp.dot(h5, w4_ref[...], preferred_element_type=f32)
                            + b4_ref[...])


def _row_spec(cols):
    return pl.BlockSpec((NB, cols), lambda i: (i, 0))


_DEG_SPEC = pl.BlockSpec((2, NB, FC), lambda i: (0, i, 0))


def _full_spec(shape):
    nd = len(shape)
    return pl.BlockSpec(shape, lambda i: (0,) * nd)


def _chunk_out_shapes():
    return [jax.ShapeDtypeStruct((N, FC), jnp.float32) for _ in range(NCHUNK)]


def _front(deg16, x, S, R, W1a, W1b, b1, W2, b2, Wg1):
    return pl.pallas_call(
        _front_body,
        grid=(GRID,),
        in_specs=[
            _DEG_SPEC, _row_spec(D_FEAT), _row_spec(64),
            _full_spec((64, SUB_REP)),
            _full_spec((D_FEAT, HID)), _full_spec((SUB_REP, HID)),
            _full_spec((1, HID)),
            _full_spec((HID, GCN_H)), _full_spec((1, GCN_H)),
            _full_spec((GCN_H, GCN_H)),
        ],
        out_specs=[_row_spec(FC)] * NCHUNK,
        out_shape=_chunk_out_shapes(),
    )(deg16, x, S, R, W1a, W1b, b1, W2, b2, Wg1)


def _mid(deg16, p, h, Wg2, bg1):
    return pl.pallas_call(
        _mid_body,
        grid=(GRID,),
        in_specs=[_DEG_SPEC] + [_row_spec(FC)] * 8 + [
            _full_spec((GCN_H, GCN_H)), _full_spec((1, GCN_H))],
        out_specs=[_row_spec(FC)] * NCHUNK,
        out_shape=_chunk_out_shapes(),
    )(deg16, *p, *h, Wg2, bg1)


def _back(deg16, p, h, bg2, W3, b3, W4, b4):
    return pl.pallas_call(
        _back_body,
        grid=(GRID,),
        in_specs=[_DEG_SPEC] + [_row_spec(FC)] * 8 + [
            _full_spec((1, GCN_H)),
            _full_spec((GCN_H, HID)), _full_spec((1, HID)),
            _full_spec((HID, OUT)), _full_spec((1, OUT))],
        out_specs=_row_spec(OUT),
        out_shape=jax.ShapeDtypeStruct((N, OUT), jnp.float32),
    )(deg16, *p, *h, bg2, W3, b3, W4, b4)


@functools.cache
def _mesh():
    return plsc.VectorSubcoreMesh(core_axis_name="c", subcore_axis_name="s")


DEG_B = 40
E_PER_TILE2 = E // (2 * TILES)
DEG_BATCHES = E_PER_TILE2 // DEG_B


def _deg_body(dst_hbm, ones_hbm, zeros_hbm, out_hbm, idx_ring, ones_v, acc,
              dsem, ssem):
    cid = lax.axis_index("c")
    sid = lax.axis_index("s")
    tbase = (cid * TILES + sid) * E_PER_TILE2
    pltpu.sync_copy(ones_hbm, ones_v)
    _per_tile_rows(sid, lambda rows, ln:
                   pltpu.sync_copy(zeros_hbm.at[pl.ds(0, ln)], acc.at[rows]))
    plsc.subcore_barrier()

    def _start_d(k, b):
        base = pl.multiple_of(tbase + k * DEG_B, 8)
        pltpu.async_copy(dst_hbm.at[pl.ds(base, DEG_B)], idx_ring[b], dsem[b])

    def _wait_d(b):
        pltpu.make_async_copy(dst_hbm.at[pl.ds(0, DEG_B)], idx_ring[b],
                              dsem[b]).wait()

    def _start_s(b):
        pltpu.async_copy(ones_v, acc.at[idx_ring[b]], ssem[b], add=True)

    def _wait_s(b):
        pltpu.make_async_copy(ones_v, acc.at[idx_ring[b]], ssem[b]).wait()

    for b in range(NBUF):
        _start_d(b, b)

    def outer(g, carry):
        for b in range(NBUF):
            k = g * NBUF + b
            bw = (b - 1) % NBUF

            @pl.when((k >= 1) & (k + NBUF - 1 < DEG_BATCHES))
            def _():
                _wait_s(bw)
                _start_d(k + NBUF - 1, bw)

            _wait_d(b)
            _start_s(b)
        return carry

    lax.fori_loop(0, DEG_BATCHES // NBUF, outer, 0)
    for b in range(NBUF):
        _wait_s(b)
    plsc.subcore_barrier()

    for core in range(2):
        @pl.when(cid == core)
        def _(core=core):
            _per_tile_rows(sid, lambda rows, ln:
                           pltpu.sync_copy(acc.at[rows], out_hbm.at[core].at[rows]))


NBUF = 5


def _scatter_body(h0_hbm, h1_hbm, h2_hbm, h3_hbm, src_hbm, dst_hbm, zeros_hbm,
                  o0_hbm, o1_hbm, o2_hbm, o3_hbm,
                  src_all, dsti, rows, acc, gsem, dsem, ssem):
    cid = lax.axis_index("c")
    sid = lax.axis_index("s")
    hw_refs = (h0_hbm, h1_hbm, h2_hbm, h3_hbm)
    out_refs = (o0_hbm, o1_hbm, o2_hbm, o3_hbm)

    ebase = pl.multiple_of(sid * E_PER_TILE, 8)
    pltpu.sync_copy(src_hbm.at[pl.ds(ebase, E_PER_TILE)], src_all)

    def _start_d(k, b):
        base = pl.multiple_of(sid * E_PER_TILE + k * EDGE_B, 8)
        pltpu.async_copy(dst_hbm.at[pl.ds(base, EDGE_B)], dsti[b], dsem[b])

    def _wait_d(b):
        pltpu.make_async_copy(dst_hbm.at[pl.ds(0, EDGE_B)], dsti[b], dsem[b]).wait()

    def _start_g(k, b, hw_ref):
        off = pl.multiple_of(k * EDGE_B, 8)
        pltpu.async_copy(hw_ref.at[src_all.at[pl.ds(off, EDGE_B)]], rows[b], gsem[b])

    def _wait_g(b, hw_ref):
        pltpu.make_async_copy(hw_ref.at[src_all.at[pl.ds(0, EDGE_B)]],
                              rows[b], gsem[b]).wait()

    def _start_s(b):
        pltpu.async_copy(rows[b], acc.at[dsti[b]], ssem[b], add=True)

    def _wait_s(b):
        pltpu.make_async_copy(rows[b], acc.at[dsti[b]], ssem[b]).wait()

    for chunk in range(NCHUNK):
        @pl.when(cid == chunk // 2)
        def _(chunk=chunk):
            hw_ref = hw_refs[chunk]
            _per_tile_rows(sid, lambda rowsl, ln:
                           pltpu.sync_copy(zeros_hbm.at[pl.ds(0, ln)], acc.at[rowsl]))
            plsc.subcore_barrier()

            for b in range(NBUF):
                _start_d(b, b)
                _start_g(b, b, hw_ref)

            def outer(g, carry):
                for b in range(NBUF):
                    k = g * NBUF + b
                    bw = (b - 1) % NBUF

                    @pl.when((k >= 1) & (k + NBUF - 1 < N_BATCH))
                    def _():
                        _wait_s(bw)
                        _start_d(k + NBUF - 1, bw)
                        _start_g(k + NBUF - 1, bw, hw_ref)

                    _wait_g(b, hw_ref)
                    _wait_d(b)
                    _start_s(b)
                return carry

            lax.fori_loop(0, N_BATCH // NBUF, outer, 0)
            for b in range(NBUF):
                _wait_s(b)
            plsc.subcore_barrier()
            _per_tile_rows(sid, lambda rowsl, ln:
                           pltpu.sync_copy(acc.at[rowsl], out_refs[chunk].at[rowsl]))


@functools.cache
def _deg_kernel_fn():
    return pl.kernel(
        _deg_body,
        mesh=_mesh(),
        out_type=jax.ShapeDtypeStruct((2, N, FC), jnp.float32),
        scratch_types=[
            [pltpu.VMEM((DEG_B,), jnp.int32) for _ in range(NBUF)],
            pltpu.VMEM((DEG_B, FC), jnp.float32),
            pltpu.VMEM_SHARED((N, FC), jnp.float32),
            [pltpu.SemaphoreType.DMA for _ in range(NBUF)],
            [pltpu.SemaphoreType.DMA for _ in range(NBUF)],
        ],
    )


@functools.cache
def _scatter_kernel_fn():
    return pl.kernel(
        _scatter_body,
        mesh=_mesh(),
        out_type=[jax.ShapeDtypeStruct((N, FC), jnp.float32) for _ in range(NCHUNK)],
        scratch_types=[
            pltpu.VMEM((E_PER_TILE,), jnp.int32),
            [pltpu.VMEM((EDGE_B,), jnp.int32) for _ in range(NBUF)],
            [pltpu.VMEM((EDGE_B, FC), jnp.float32) for _ in range(NBUF)],
            pltpu.VMEM_SHARED((N, FC), jnp.float32),
            [pltpu.SemaphoreType.DMA for _ in range(NBUF)],
            [pltpu.SemaphoreType.DMA for _ in range(NBUF)],
            [pltpu.SemaphoreType.DMA for _ in range(NBUF)],
        ],
    )


def _deg_kernel(dst, onesfc, zerosfc):
    return _deg_kernel_fn()(dst, onesfc, zerosfc)


def _scatter_kernel(h0, h1, h2, h3, src, dst, zerosfc):
    return _scatter_kernel_fn()(h0, h1, h2, h3, src, dst, zerosfc)


def kernel(x, edge_index, S, R, W1, b1, W2, b2, Wg1, bg1, Wg2, bg2, W3, b3, W4, b4):
    f32 = jnp.float32
    src = edge_index[0].astype(jnp.int32)
    dst = edge_index[1].astype(jnp.int32)
    onesfc = jnp.ones((DEG_B, FC), f32)
    zerosfc = jnp.zeros((ROWS_MAIN, FC), f32)
    b1r = b1.reshape(1, -1)
    b2r = b2.reshape(1, -1)
    bg1r = bg1.reshape(1, -1)
    bg2r = bg2.reshape(1, -1)
    b3r = b3.reshape(1, -1)
    b4r = b4.reshape(1, -1)

    deg16 = _deg_kernel(dst, onesfc, zerosfc)
    h1c = _front(deg16, x, S, R, W1[:D_FEAT], W1[D_FEAT:], b1r, W2, b2r, Wg1)
    p1c = _scatter_kernel(*h1c, src, dst, zerosfc)
    h2c = _mid(deg16, p1c, h1c, Wg2, bg1r)
    p2c = _scatter_kernel(*h2c, src, dst, zerosfc)
    out = _back(deg16, p2c, h2c, bg2r, W3, b3r, W4, b4r)
    return out


# speedup vs baseline: 16.5536x; 1.0037x over previous
import functools

import jax
import jax.numpy as jnp
from jax import lax
from jax.experimental import pallas as pl
from jax.experimental.pallas import tpu as pltpu
from jax.experimental.pallas import tpu_sc as plsc

N = 10000
E = 160000
D_FEAT = 256
SUB_REP = 256
HID = 1024
GCN_H = 512
OUT = 128

NB = 1000
GRID = N // NB
NCHUNK = 4
FC = GCN_H // NCHUNK
TILES = 16
E_PER_TILE = E // TILES
EDGE_B = 40
N_BATCH = E_PER_TILE // EDGE_B
ROWS_MAIN = 640
ROWS_LAST = N - (TILES - 1) * ROWS_MAIN


def _per_tile_rows(sid, fn):
    @pl.when(sid < TILES - 1)
    def _():
        fn(pl.ds(pl.multiple_of(sid * ROWS_MAIN, 8), ROWS_MAIN), ROWS_MAIN)

    @pl.when(sid == TILES - 1)
    def _():
        fn(pl.ds((TILES - 1) * ROWS_MAIN, ROWS_LAST), ROWS_LAST)


def _dinv(deg_ref):
    return lax.rsqrt(deg_ref[0, :, :1] + deg_ref[1, :, :1] + 1.0)


def _bdot(a, w_ref):
    return jnp.dot(a.astype(jnp.bfloat16), w_ref[...],
                   preferred_element_type=jnp.float32)


def _front_body(deg_ref, x_ref, s_ref, r_ref, w1a_ref, w1b_ref, b1_ref,
                w2_ref, b2_ref, wg1_ref, o0, o1, o2, o3):
    sr = _bdot(s_ref[...], r_ref)
    t = _bdot(x_ref[...], w1a_ref) + _bdot(sr, w1b_ref) + b1_ref[...]
    h1 = jnp.tanh(t)
    h2 = jnp.tanh(_bdot(h1, w2_ref) + b2_ref[...])
    hw = _bdot(h2, wg1_ref)
    dinv = _dinv(deg_ref)
    hws = hw * dinv
    o0[...] = hws[:, 0 * FC:1 * FC]
    o1[...] = hws[:, 1 * FC:2 * FC]
    o2[...] = hws[:, 2 * FC:3 * FC]
    o3[...] = hws[:, 3 * FC:4 * FC]


def _mid_body(deg_ref, p0, p1, p2, p3, h0, h1, h2, h3, wg_ref, bg_ref,
              o0, o1, o2, o3):
    p = jnp.concatenate([p0[...], p1[...], p2[...], p3[...]], axis=1)
    hprev = jnp.concatenate([h0[...], h1[...], h2[...], h3[...]], axis=1)
    dinv = _dinv(deg_ref)
    h = jnp.maximum(dinv * (p + hprev) + bg_ref[...], 0.0)
    hw = _bdot(h, wg_ref) * dinv
    o0[...] = hw[:, 0 * FC:1 * FC]
    o1[...] = hw[:, 1 * FC:2 * FC]
    o2[...] = hw[:, 2 * FC:3 * FC]
    o3[...] = hw[:, 3 * FC:4 * FC]


def _back_body(deg_ref, p0, p1, p2, p3, h0, h1, h2, h3, bg_ref,
               w3_ref, b3_ref, w4_ref, b4_ref, out_ref):
    p = jnp.concatenate([p0[...], p1[...], p2[...], p3[...]], axis=1)
    hprev = jnp.concatenate([h0[...], h1[...], h2[...], h3[...]], axis=1)
    dinv = _dinv(deg_ref)
    h = jnp.maximum(dinv * (p + hprev) + bg_ref[...], 0.0)
    h5 = jnp.tanh(_bdot(h, w3_ref) + b3_ref[...])
    out_ref[...] = jnp.tanh(_bdot(h5, w4_ref) + b4_ref[...])


def _row_spec(cols):
    return pl.BlockSpec((NB, cols), lambda i: (i, 0))


_DEG_SPEC = pl.BlockSpec((2, NB, FC), lambda i: (0, i, 0))


def _full_spec(shape):
    nd = len(shape)
    return pl.BlockSpec(shape, lambda i: (0,) * nd)


def _chunk_out_shapes():
    return [jax.ShapeDtypeStruct((N, FC), jnp.float32) for _ in range(NCHUNK)]


def _front(deg16, x, S, R, W1a, W1b, b1, W2, b2, Wg1):
    return pl.pallas_call(
        _front_body,
        grid=(GRID,),
        in_specs=[
            _DEG_SPEC, _row_spec(D_FEAT), _row_spec(64),
            _full_spec((64, SUB_REP)),
            _full_spec((D_FEAT, HID)), _full_spec((SUB_REP, HID)),
            _full_spec((1, HID)),
            _full_spec((HID, GCN_H)), _full_spec((1, GCN_H)),
            _full_spec((GCN_H, GCN_H)),
        ],
        out_specs=[_row_spec(FC)] * NCHUNK,
        out_shape=_chunk_out_shapes(),
    )(deg16, x, S, R, W1a, W1b, b1, W2, b2, Wg1)


def _mid(deg16, p, h, Wg2, bg1):
    return pl.pallas_call(
        _mid_body,
        grid=(GRID,),
        in_specs=[_DEG_SPEC] + [_row_spec(FC)] * 8 + [
            _full_spec((GCN_H, GCN_H)), _full_spec((1, GCN_H))],
        out_specs=[_row_spec(FC)] * NCHUNK,
        out_shape=_chunk_out_shapes(),
    )(deg16, *p, *h, Wg2, bg1)


def _back(deg16, p, h, bg2, W3, b3, W4, b4):
    return pl.pallas_call(
        _back_body,
        grid=(GRID,),
        in_specs=[_DEG_SPEC] + [_row_spec(FC)] * 8 + [
            _full_spec((1, GCN_H)),
            _full_spec((GCN_H, HID)), _full_spec((1, HID)),
            _full_spec((HID, OUT)), _full_spec((1, OUT))],
        out_specs=_row_spec(OUT),
        out_shape=jax.ShapeDtypeStruct((N, OUT), jnp.float32),
    )(deg16, *p, *h, bg2, W3, b3, W4, b4)


@functools.cache
def _mesh():
    return plsc.VectorSubcoreMesh(core_axis_name="c", subcore_axis_name="s")


DEG_B = 40
E_PER_TILE2 = E // (2 * TILES)
DEG_BATCHES = E_PER_TILE2 // DEG_B


def _deg_body(dst_hbm, ones_hbm, zeros_hbm, out_hbm, idx_ring, ones_v, acc,
              dsem, ssem):
    cid = lax.axis_index("c")
    sid = lax.axis_index("s")
    tbase = (cid * TILES + sid) * E_PER_TILE2
    pltpu.sync_copy(ones_hbm, ones_v)
    _per_tile_rows(sid, lambda rows, ln:
                   pltpu.sync_copy(zeros_hbm.at[pl.ds(0, ln)], acc.at[rows]))
    plsc.subcore_barrier()

    def _start_d(k, b):
        base = pl.multiple_of(tbase + k * DEG_B, 8)
        pltpu.async_copy(dst_hbm.at[pl.ds(base, DEG_B)], idx_ring[b], dsem[b])

    def _wait_d(b):
        pltpu.make_async_copy(dst_hbm.at[pl.ds(0, DEG_B)], idx_ring[b],
                              dsem[b]).wait()

    def _start_s(b):
        pltpu.async_copy(ones_v, acc.at[idx_ring[b]], ssem[b], add=True)

    def _wait_s(b):
        pltpu.make_async_copy(ones_v, acc.at[idx_ring[b]], ssem[b]).wait()

    for b in range(NBUF):
        _start_d(b, b)

    def outer(g, carry):
        for b in range(NBUF):
            k = g * NBUF + b
            bw = (b - 1) % NBUF

            @pl.when((k >= 1) & (k + NBUF - 1 < DEG_BATCHES))
            def _():
                _wait_s(bw)
                _start_d(k + NBUF - 1, bw)

            _wait_d(b)
            _start_s(b)
        return carry

    lax.fori_loop(0, DEG_BATCHES // NBUF, outer, 0)
    for b in range(NBUF):
        _wait_s(b)
    plsc.subcore_barrier()

    for core in range(2):
        @pl.when(cid == core)
        def _(core=core):
            _per_tile_rows(sid, lambda rows, ln:
                           pltpu.sync_copy(acc.at[rows], out_hbm.at[core].at[rows]))


NBUF = 5


def _scatter_body(h0_hbm, h1_hbm, h2_hbm, h3_hbm, src_hbm, dst_hbm, zeros_hbm,
                  o0_hbm, o1_hbm, o2_hbm, o3_hbm,
                  src_all, dsti, rows, acc, gsem, dsem, ssem):
    cid = lax.axis_index("c")
    sid = lax.axis_index("s")
    hw_refs = (h0_hbm, h1_hbm, h2_hbm, h3_hbm)
    out_refs = (o0_hbm, o1_hbm, o2_hbm, o3_hbm)

    ebase = pl.multiple_of(sid * E_PER_TILE, 8)
    pltpu.sync_copy(src_hbm.at[pl.ds(ebase, E_PER_TILE)], src_all)

    def _start_d(k, b):
        base = pl.multiple_of(sid * E_PER_TILE + k * EDGE_B, 8)
        pltpu.async_copy(dst_hbm.at[pl.ds(base, EDGE_B)], dsti[b], dsem[b])

    def _wait_d(b):
        pltpu.make_async_copy(dst_hbm.at[pl.ds(0, EDGE_B)], dsti[b], dsem[b]).wait()

    def _start_g(k, b, hw_ref):
        off = pl.multiple_of(k * EDGE_B, 8)
        pltpu.async_copy(hw_ref.at[src_all.at[pl.ds(off, EDGE_B)]], rows[b], gsem[b])

    def _wait_g(b, hw_ref):
        pltpu.make_async_copy(hw_ref.at[src_all.at[pl.ds(0, EDGE_B)]],
                              rows[b], gsem[b]).wait()

    def _start_s(b):
        pltpu.async_copy(rows[b], acc.at[dsti[b]], ssem[b], add=True)

    def _wait_s(b):
        pltpu.make_async_copy(rows[b], acc.at[dsti[b]], ssem[b]).wait()

    for chunk in range(NCHUNK):
        @pl.when(cid == chunk // 2)
        def _(chunk=chunk):
            hw_ref = hw_refs[chunk]
            _per_tile_rows(sid, lambda rowsl, ln:
                           pltpu.sync_copy(zeros_hbm.at[pl.ds(0, ln)], acc.at[rowsl]))
            plsc.subcore_barrier()

            for b in range(NBUF):
                _start_d(b, b)
                _start_g(b, b, hw_ref)

            def outer(g, carry):
                for b in range(NBUF):
                    k = g * NBUF + b
                    bw = (b - 1) % NBUF

                    @pl.when((k >= 1) & (k + NBUF - 1 < N_BATCH))
                    def _():
                        _wait_s(bw)
                        _start_d(k + NBUF - 1, bw)
                        _start_g(k + NBUF - 1, bw, hw_ref)

                    _wait_g(b, hw_ref)
                    _wait_d(b)
                    _start_s(b)
                return carry

            lax.fori_loop(0, N_BATCH // NBUF, outer, 0)
            for b in range(NBUF):
                _wait_s(b)
            plsc.subcore_barrier()
            _per_tile_rows(sid, lambda rowsl, ln:
                           pltpu.sync_copy(acc.at[rowsl], out_refs[chunk].at[rowsl]))


@functools.cache
def _deg_kernel_fn():
    return pl.kernel(
        _deg_body,
        mesh=_mesh(),
        out_type=jax.ShapeDtypeStruct((2, N, FC), jnp.float32),
        scratch_types=[
            [pltpu.VMEM((DEG_B,), jnp.int32) for _ in range(NBUF)],
            pltpu.VMEM((DEG_B, FC), jnp.float32),
            pltpu.VMEM_SHARED((N, FC), jnp.float32),
            [pltpu.SemaphoreType.DMA for _ in range(NBUF)],
            [pltpu.SemaphoreType.DMA for _ in range(NBUF)],
        ],
    )


@functools.cache
def _scatter_kernel_fn():
    return pl.kernel(
        _scatter_body,
        mesh=_mesh(),
        out_type=[jax.ShapeDtypeStruct((N, FC), jnp.float32) for _ in range(NCHUNK)],
        scratch_types=[
            pltpu.VMEM((E_PER_TILE,), jnp.int32),
            [pltpu.VMEM((EDGE_B,), jnp.int32) for _ in range(NBUF)],
            [pltpu.VMEM((EDGE_B, FC), jnp.float32) for _ in range(NBUF)],
            pltpu.VMEM_SHARED((N, FC), jnp.float32),
            [pltpu.SemaphoreType.DMA for _ in range(NBUF)],
            [pltpu.SemaphoreType.DMA for _ in range(NBUF)],
            [pltpu.SemaphoreType.DMA for _ in range(NBUF)],
        ],
    )


def _deg_kernel(dst, onesfc, zerosfc):
    return _deg_kernel_fn()(dst, onesfc, zerosfc)


def _scatter_kernel(h0, h1, h2, h3, src, dst, zerosfc):
    return _scatter_kernel_fn()(h0, h1, h2, h3, src, dst, zerosfc)


def kernel(x, edge_index, S, R, W1, b1, W2, b2, Wg1, bg1, Wg2, bg2, W3, b3, W4, b4):
    f32 = jnp.float32
    src = edge_index[0].astype(jnp.int32)
    dst = edge_index[1].astype(jnp.int32)
    onesfc = jnp.ones((DEG_B, FC), f32)
    zerosfc = jnp.zeros((ROWS_MAIN, FC), f32)
    bf16 = jnp.bfloat16
    W1a = W1[:D_FEAT].astype(bf16)
    W1b = W1[D_FEAT:].astype(bf16)
    W2c = W2.astype(bf16)
    Wg1c = Wg1.astype(bf16)
    Wg2c = Wg2.astype(bf16)
    W3c = W3.astype(bf16)
    W4c = W4.astype(bf16)
    Rc = R.astype(bf16)
    b1r = b1.reshape(1, -1)
    b2r = b2.reshape(1, -1)
    bg1r = bg1.reshape(1, -1)
    bg2r = bg2.reshape(1, -1)
    b3r = b3.reshape(1, -1)
    b4r = b4.reshape(1, -1)

    deg16 = _deg_kernel(dst, onesfc, zerosfc)
    h1c = _front(deg16, x, S, Rc, W1a, W1b, b1r, W2c, b2r, Wg1c)
    p1c = _scatter_kernel(*h1c, src, dst, zerosfc)
    h2c = _mid(deg16, p1c, h1c, Wg2c, bg1r)
    p2c = _scatter_kernel(*h2c, src, dst, zerosfc)
    out = _back(deg16, p2c, h2c, bg2r, W3c, b3r, W4c, b4r)
    return out


# speedup vs baseline: 17.0643x; 1.0309x over previous
import functools

import jax
import jax.numpy as jnp
from jax import lax
from jax.experimental import pallas as pl
from jax.experimental.pallas import tpu as pltpu
from jax.experimental.pallas import tpu_sc as plsc

N = 10000
E = 160000
D_FEAT = 256
SUB_REP = 256
HID = 1024
GCN_H = 512
OUT = 128

NB = 1000
GRID = N // NB
NCHUNK = 4
FC = GCN_H // NCHUNK
TILES = 16
E_PER_TILE = E // TILES
EDGE_B = 40
N_BATCH = E_PER_TILE // EDGE_B
ROWS_MAIN = 640
ROWS_LAST = N - (TILES - 1) * ROWS_MAIN


def _per_tile_rows(sid, fn):
    @pl.when(sid < TILES - 1)
    def _():
        fn(pl.ds(pl.multiple_of(sid * ROWS_MAIN, 8), ROWS_MAIN), ROWS_MAIN)

    @pl.when(sid == TILES - 1)
    def _():
        fn(pl.ds((TILES - 1) * ROWS_MAIN, ROWS_LAST), ROWS_LAST)


def _dinv(deg_ref):
    return lax.rsqrt(deg_ref[0, :, :1] + deg_ref[1, :, :1] + 1.0)


def _bdot(a, w_ref):
    return jnp.dot(a.astype(jnp.bfloat16), w_ref[...],
                   preferred_element_type=jnp.float32)


def _front_body(x_ref, s_ref, r_ref, w1a_ref, w1b_ref, b1_ref,
                w2_ref, b2_ref, wg1_ref, hw_out):
    sr = _bdot(s_ref[...], r_ref)
    t = _bdot(x_ref[...], w1a_ref) + _bdot(sr, w1b_ref) + b1_ref[...]
    h1 = jnp.tanh(t)
    h2 = jnp.tanh(_bdot(h1, w2_ref) + b2_ref[...])
    hw_out[...] = _bdot(h2, wg1_ref)


def _scale_body(deg_ref, hw_ref, o0, o1, o2, o3, dv_out):
    dinv = _dinv(deg_ref)
    hws = hw_ref[...] * dinv
    o0[...] = hws[:, 0 * FC:1 * FC]
    o1[...] = hws[:, 1 * FC:2 * FC]
    o2[...] = hws[:, 2 * FC:3 * FC]
    o3[...] = hws[:, 3 * FC:4 * FC]
    dv_out[...] = jnp.broadcast_to(dinv, (NB, FC))


def _mid_body(dv_ref, p0, p1, p2, p3, h0, h1, h2, h3, wg_ref, bg_ref,
              o0, o1, o2, o3):
    p = jnp.concatenate([p0[...], p1[...], p2[...], p3[...]], axis=1)
    hprev = jnp.concatenate([h0[...], h1[...], h2[...], h3[...]], axis=1)
    dinv = dv_ref[:, :1]
    h = jnp.maximum(dinv * (p + hprev) + bg_ref[...], 0.0)
    hw = _bdot(h, wg_ref) * dinv
    o0[...] = hw[:, 0 * FC:1 * FC]
    o1[...] = hw[:, 1 * FC:2 * FC]
    o2[...] = hw[:, 2 * FC:3 * FC]
    o3[...] = hw[:, 3 * FC:4 * FC]


def _back_body(dv_ref, p0, p1, p2, p3, h0, h1, h2, h3, bg_ref,
               w3_ref, b3_ref, w4_ref, b4_ref, out_ref):
    p = jnp.concatenate([p0[...], p1[...], p2[...], p3[...]], axis=1)
    hprev = jnp.concatenate([h0[...], h1[...], h2[...], h3[...]], axis=1)
    dinv = dv_ref[:, :1]
    h = jnp.maximum(dinv * (p + hprev) + bg_ref[...], 0.0)
    h5 = jnp.tanh(_bdot(h, w3_ref) + b3_ref[...])
    out_ref[...] = jnp.tanh(_bdot(h5, w4_ref) + b4_ref[...])


def _row_spec(cols):
    return pl.BlockSpec((NB, cols), lambda i: (i, 0))


_DEG_SPEC = pl.BlockSpec((2, NB, FC), lambda i: (0, i, 0))


def _full_spec(shape):
    nd = len(shape)
    return pl.BlockSpec(shape, lambda i: (0,) * nd)


def _chunk_out_shapes():
    return [jax.ShapeDtypeStruct((N, FC), jnp.float32) for _ in range(NCHUNK)]


def _front(x, S, R, W1a, W1b, b1, W2, b2, Wg1):
    return pl.pallas_call(
        _front_body,
        grid=(GRID,),
        in_specs=[
            _row_spec(D_FEAT), _row_spec(64),
            _full_spec((64, SUB_REP)),
            _full_spec((D_FEAT, HID)), _full_spec((SUB_REP, HID)),
            _full_spec((1, HID)),
            _full_spec((HID, GCN_H)), _full_spec((1, GCN_H)),
            _full_spec((GCN_H, GCN_H)),
        ],
        out_specs=_row_spec(GCN_H),
        out_shape=jax.ShapeDtypeStruct((N, GCN_H), jnp.float32),
    )(x, S, R, W1a, W1b, b1, W2, b2, Wg1)


def _scale(deg16, hw):
    return pl.pallas_call(
        _scale_body,
        grid=(GRID,),
        in_specs=[_DEG_SPEC, _row_spec(GCN_H)],
        out_specs=[_row_spec(FC)] * (NCHUNK + 1),
        out_shape=_chunk_out_shapes()
        + [jax.ShapeDtypeStruct((N, FC), jnp.float32)],
    )(deg16, hw)


def _mid(dv, p, h, Wg2, bg1):
    return pl.pallas_call(
        _mid_body,
        grid=(GRID,),
        in_specs=[_row_spec(FC)] + [_row_spec(FC)] * 8 + [
            _full_spec((GCN_H, GCN_H)), _full_spec((1, GCN_H))],
        out_specs=[_row_spec(FC)] * NCHUNK,
        out_shape=_chunk_out_shapes(),
    )(dv, *p, *h, Wg2, bg1)


def _back(dv, p, h, bg2, W3, b3, W4, b4):
    return pl.pallas_call(
        _back_body,
        grid=(GRID,),
        in_specs=[_row_spec(FC)] + [_row_spec(FC)] * 8 + [
            _full_spec((1, GCN_H)),
            _full_spec((GCN_H, HID)), _full_spec((1, HID)),
            _full_spec((HID, OUT)), _full_spec((1, OUT))],
        out_specs=_row_spec(OUT),
        out_shape=jax.ShapeDtypeStruct((N, OUT), jnp.float32),
    )(dv, *p, *h, bg2, W3, b3, W4, b4)


@functools.cache
def _mesh():
    return plsc.VectorSubcoreMesh(core_axis_name="c", subcore_axis_name="s")


DEG_B = 40
E_PER_TILE2 = E // (2 * TILES)
DEG_BATCHES = E_PER_TILE2 // DEG_B


def _deg_body(dst_hbm, ones_hbm, zeros_hbm, out_hbm, idx_ring, ones_v, acc,
              dsem, ssem):
    cid = lax.axis_index("c")
    sid = lax.axis_index("s")
    tbase = (cid * TILES + sid) * E_PER_TILE2
    pltpu.sync_copy(ones_hbm, ones_v)
    _per_tile_rows(sid, lambda rows, ln:
                   pltpu.sync_copy(zeros_hbm.at[pl.ds(0, ln)], acc.at[rows]))
    plsc.subcore_barrier()

    def _start_d(k, b):
        base = pl.multiple_of(tbase + k * DEG_B, 8)
        pltpu.async_copy(dst_hbm.at[pl.ds(base, DEG_B)], idx_ring[b], dsem[b])

    def _wait_d(b):
        pltpu.make_async_copy(dst_hbm.at[pl.ds(0, DEG_B)], idx_ring[b],
                              dsem[b]).wait()

    def _start_s(b):
        pltpu.async_copy(ones_v, acc.at[idx_ring[b]], ssem[b], add=True)

    def _wait_s(b):
        pltpu.make_async_copy(ones_v, acc.at[idx_ring[b]], ssem[b]).wait()

    for b in range(NBUF):
        _start_d(b, b)

    def outer(g, carry):
        for b in range(NBUF):
            k = g * NBUF + b
            bw = (b - 1) % NBUF

            @pl.when((k >= 1) & (k + NBUF - 1 < DEG_BATCHES))
            def _():
                _wait_s(bw)
                _start_d(k + NBUF - 1, bw)

            _wait_d(b)
            _start_s(b)
        return carry

    lax.fori_loop(0, DEG_BATCHES // NBUF, outer, 0)
    for b in range(NBUF):
        _wait_s(b)
    plsc.subcore_barrier()

    for core in range(2):
        @pl.when(cid == core)
        def _(core=core):
            _per_tile_rows(sid, lambda rows, ln:
                           pltpu.sync_copy(acc.at[rows], out_hbm.at[core].at[rows]))


NBUF = 5


def _scatter_body(h0_hbm, h1_hbm, h2_hbm, h3_hbm, src_hbm, dst_hbm, zeros_hbm,
                  o0_hbm, o1_hbm, o2_hbm, o3_hbm,
                  src_all, dsti, rows, acc, gsem, dsem, ssem):
    cid = lax.axis_index("c")
    sid = lax.axis_index("s")
    hw_refs = (h0_hbm, h1_hbm, h2_hbm, h3_hbm)
    out_refs = (o0_hbm, o1_hbm, o2_hbm, o3_hbm)

    ebase = pl.multiple_of(sid * E_PER_TILE, 8)
    pltpu.sync_copy(src_hbm.at[pl.ds(ebase, E_PER_TILE)], src_all)

    def _start_d(k, b):
        base = pl.multiple_of(sid * E_PER_TILE + k * EDGE_B, 8)
        pltpu.async_copy(dst_hbm.at[pl.ds(base, EDGE_B)], dsti[b], dsem[b])

    def _wait_d(b):
        pltpu.make_async_copy(dst_hbm.at[pl.ds(0, EDGE_B)], dsti[b], dsem[b]).wait()

    def _start_g(k, b, hw_ref):
        off = pl.multiple_of(k * EDGE_B, 8)
        pltpu.async_copy(hw_ref.at[src_all.at[pl.ds(off, EDGE_B)]], rows[b], gsem[b])

    def _wait_g(b, hw_ref):
        pltpu.make_async_copy(hw_ref.at[src_all.at[pl.ds(0, EDGE_B)]],
                              rows[b], gsem[b]).wait()

    def _start_s(b):
        pltpu.async_copy(rows[b], acc.at[dsti[b]], ssem[b], add=True)

    def _wait_s(b):
        pltpu.make_async_copy(rows[b], acc.at[dsti[b]], ssem[b]).wait()

    for chunk in range(NCHUNK):
        @pl.when(cid == chunk // 2)
        def _(chunk=chunk):
            hw_ref = hw_refs[chunk]
            _per_tile_rows(sid, lambda rowsl, ln:
                           pltpu.sync_copy(zeros_hbm.at[pl.ds(0, ln)], acc.at[rowsl]))
            plsc.subcore_barrier()

            for b in range(NBUF):
                _start_d(b, b)
                _start_g(b, b, hw_ref)

            def outer(g, carry):
                for b in range(NBUF):
                    k = g * NBUF + b
                    bw = (b - 1) % NBUF

                    @pl.when((k >= 1) & (k + NBUF - 1 < N_BATCH))
                    def _():
                        _wait_s(bw)
                        _start_d(k + NBUF - 1, bw)
                        _start_g(k + NBUF - 1, bw, hw_ref)

                    _wait_g(b, hw_ref)
                    _wait_d(b)
                    _start_s(b)
                return carry

            lax.fori_loop(0, N_BATCH // NBUF, outer, 0)
            for b in range(NBUF):
                _wait_s(b)
            plsc.subcore_barrier()
            _per_tile_rows(sid, lambda rowsl, ln:
                           pltpu.sync_copy(acc.at[rowsl], out_refs[chunk].at[rowsl]))


@functools.cache
def _deg_kernel_fn():
    return pl.kernel(
        _deg_body,
        mesh=_mesh(),
        out_type=jax.ShapeDtypeStruct((2, N, FC), jnp.float32),
        scratch_types=[
            [pltpu.VMEM((DEG_B,), jnp.int32) for _ in range(NBUF)],
            pltpu.VMEM((DEG_B, FC), jnp.float32),
            pltpu.VMEM_SHARED((N, FC), jnp.float32),
            [pltpu.SemaphoreType.DMA for _ in range(NBUF)],
            [pltpu.SemaphoreType.DMA for _ in range(NBUF)],
        ],
    )


@functools.cache
def _scatter_kernel_fn():
    return pl.kernel(
        _scatter_body,
        mesh=_mesh(),
        out_type=[jax.ShapeDtypeStruct((N, FC), jnp.float32) for _ in range(NCHUNK)],
        scratch_types=[
            pltpu.VMEM((E_PER_TILE,), jnp.int32),
            [pltpu.VMEM((EDGE_B,), jnp.int32) for _ in range(NBUF)],
            [pltpu.VMEM((EDGE_B, FC), jnp.float32) for _ in range(NBUF)],
            pltpu.VMEM_SHARED((N, FC), jnp.float32),
            [pltpu.SemaphoreType.DMA for _ in range(NBUF)],
            [pltpu.SemaphoreType.DMA for _ in range(NBUF)],
            [pltpu.SemaphoreType.DMA for _ in range(NBUF)],
        ],
    )


def _deg_kernel(dst, onesfc, zerosfc):
    return _deg_kernel_fn()(dst, onesfc, zerosfc)


def _scatter_kernel(h0, h1, h2, h3, src, dst, zerosfc):
    return _scatter_kernel_fn()(h0, h1, h2, h3, src, dst, zerosfc)


def kernel(x, edge_index, S, R, W1, b1, W2, b2, Wg1, bg1, Wg2, bg2, W3, b3, W4, b4):
    f32 = jnp.float32
    src = edge_index[0].astype(jnp.int32)
    dst = edge_index[1].astype(jnp.int32)
    onesfc = jnp.ones((DEG_B, FC), f32)
    zerosfc = jnp.zeros((ROWS_MAIN, FC), f32)
    bf16 = jnp.bfloat16
    W1a = W1[:D_FEAT].astype(bf16)
    W1b = W1[D_FEAT:].astype(bf16)
    W2c = W2.astype(bf16)
    Wg1c = Wg1.astype(bf16)
    Wg2c = Wg2.astype(bf16)
    W3c = W3.astype(bf16)
    W4c = W4.astype(bf16)
    Rc = R.astype(bf16)
    b1r = b1.reshape(1, -1)
    b2r = b2.reshape(1, -1)
    bg1r = bg1.reshape(1, -1)
    bg2r = bg2.reshape(1, -1)
    b3r = b3.reshape(1, -1)
    b4r = b4.reshape(1, -1)

    deg16 = _deg_kernel(dst, onesfc, zerosfc)
    hw1 = _front(x, S, Rc, W1a, W1b, b1r, W2c, b2r, Wg1c)
    *h1c, dv = _scale(deg16, hw1)
    p1c = _scatter_kernel(*h1c, src, dst, zerosfc)
    h2c = _mid(dv, p1c, h1c, Wg2c, bg1r)
    p2c = _scatter_kernel(*h2c, src, dst, zerosfc)
    out = _back(dv, p2c, h2c, bg2r, W3c, b3r, W4c, b4r)
    return out
